# baseline (speedup 1.0000x reference)
import functools

import jax
import jax.numpy as jnp
from jax import lax
from jax.experimental import pallas as pl
from jax.experimental.pallas import tpu as pltpu
from jax.experimental.pallas import tpu_sc as plsc

_N = 10000
_E = 160000
_G = 64
_D = 256
_HALF = 128
_R = 1000
_GRID = _N // _R

_NP = 10240
_TILES = 16
_RPT = _NP // _TILES
_CH = 80
_EPT = _NP
_CPT = _EPT // _CH
_NBUF = 3
_ROUNDS = (_CPT - 2) // _NBUF
_CHD = 128
_EPTD = 5120
_CPTD = _EPTD // _CHD


def _first_body(x_ref, degt_ref, w_ref, g_ref, dinv_ref):
    d2 = degt_ref[...]
    deg = 1.0 + d2[:, 0:1] + d2[:, 1:2]
    dinv = lax.rsqrt(deg)
    gn = dinv * jnp.dot(x_ref[...], w_ref[...], preferred_element_type=jnp.float32)
    g_ref[0] = gn[:, :_HALF]
    g_ref[1] = gn[:, _HALF:]
    dinv_ref[...] = dinv


def _first_layer(x, degt, w1):
    return pl.pallas_call(
        _first_body,
        grid=(_GRID,),
        in_specs=[
            pl.BlockSpec((_R, _D), lambda i: (i, 0)),
            pl.BlockSpec((_R, 2), lambda i: (i, 0)),
            pl.BlockSpec((_D, _D), lambda i: (0, 0)),
        ],
        out_specs=[
            pl.BlockSpec((2, _R, _HALF), lambda i: (0, i, 0)),
            pl.BlockSpec((_R, 1), lambda i: (i, 0)),
        ],
        out_shape=[
            jax.ShapeDtypeStruct((2, _NP, _HALF), jnp.float32),
            jax.ShapeDtypeStruct((_N, 1), jnp.float32),
        ],
    )(x, degt, w1)


def _mid_body(s_ref, g_ref, b_ref, dinv_ref, w_ref, *rest, has_res, act_out):
    if has_res:
        res_ref = rest[0]
        rest = rest[1:]
    gn_ref = rest[0]
    dinv = dinv_ref[...]
    s_cat = jnp.concatenate([s_ref[0], s_ref[1]], axis=1)
    g_cat = jnp.concatenate([g_ref[0], g_ref[1]], axis=1)
    h = dinv * (s_cat + g_cat) + b_ref[...][None, :]
    act = jnp.maximum(h, 0.0)
    if has_res:
        act = act + res_ref[...]
    if act_out:
        rest[1][...] = act
    gn = dinv * jnp.dot(act, w_ref[...], preferred_element_type=jnp.float32)
    gn_ref[0] = gn[:, :_HALF]
    gn_ref[1] = gn[:, _HALF:]


def _mid_layer(s, g, b, dinv, w, res=None, act_out=False):
    has_res = res is not None
    in_specs = [
        pl.BlockSpec((2, _R, _HALF), lambda i: (0, i, 0)),
        pl.BlockSpec((2, _R, _HALF), lambda i: (0, i, 0)),
        pl.BlockSpec((_D,), lambda i: (0,)),
        pl.BlockSpec((_R, 1), lambda i: (i, 0)),
        pl.BlockSpec((_D, _D), lambda i: (0, 0)),
    ]
    args = [s, g, b, dinv, w]
    if has_res:
        in_specs.append(pl.BlockSpec((_R, _D), lambda i: (i, 0)))
        args.append(res)
    out_specs = [pl.BlockSpec((2, _R, _HALF), lambda i: (0, i, 0))]
    out_shape = [jax.ShapeDtypeStruct((2, _NP, _HALF), jnp.float32)]
    if act_out:
        out_specs.append(pl.BlockSpec((_R, _D), lambda i: (i, 0)))
        out_shape.append(jax.ShapeDtypeStruct((_N, _D), jnp.float32))
    out = pl.pallas_call(
        functools.partial(_mid_body, has_res=has_res, act_out=act_out),
        grid=(_GRID,),
        in_specs=in_specs,
        out_specs=out_specs,
        out_shape=out_shape,
    )(*args)
    return out if act_out else out[0]


def _pool_body(s_ref, g_ref, b_ref, dinv_ref, batch_ref, sums_ref, cnts_ref):
    i = pl.program_id(0)
    s_cat = jnp.concatenate([s_ref[0], s_ref[1]], axis=1)
    g_cat = jnp.concatenate([g_ref[0], g_ref[1]], axis=1)
    h = dinv_ref[...] * (s_cat + g_cat) + b_ref[...][None, :]
    gid = lax.broadcasted_iota(jnp.int32, (_R, _G), 1)
    mask = (batch_ref[...] == gid).astype(jnp.float32)
    part = lax.dot_general(mask, h, (((0,), (0,)), ((), ())),
                           preferred_element_type=jnp.float32)
    cpart = lax.dot_general(mask, jnp.ones((_R, 1), jnp.float32),
                            (((0,), (0,)), ((), ())),
                            preferred_element_type=jnp.float32)

    @pl.when(i == 0)
    def _init():
        sums_ref[...] = jnp.zeros_like(sums_ref)
        cnts_ref[...] = jnp.zeros_like(cnts_ref)

    sums_ref[...] += part
    cnts_ref[...] += cpart


def _pool(s, g, b, dinv, batch2d):
    return pl.pallas_call(
        _pool_body,
        grid=(_GRID,),
        in_specs=[
            pl.BlockSpec((2, _R, _HALF), lambda i: (0, i, 0)),
            pl.BlockSpec((2, _R, _HALF), lambda i: (0, i, 0)),
            pl.BlockSpec((_D,), lambda i: (0,)),
            pl.BlockSpec((_R, 1), lambda i: (i, 0)),
            pl.BlockSpec((_R, 1), lambda i: (i, 0)),
        ],
        out_specs=[
            pl.BlockSpec((_G, _D), lambda i: (0, 0)),
            pl.BlockSpec((_G, 1), lambda i: (0, 0)),
        ],
        out_shape=[
            jax.ShapeDtypeStruct((_G, _D), jnp.float32),
            jax.ShapeDtypeStruct((_G, 1), jnp.float32),
        ],
    )(s, g, b, dinv, batch2d)


def _mlp_body(sums_ref, cnts_ref, w1_ref, b1_ref, w15_ref, b15_ref, w2_ref,
              b2_ref, out_ref):
    pooled = sums_ref[...] / jnp.maximum(cnts_ref[...], 1.0)
    o = jnp.maximum(jnp.dot(pooled, w1_ref[...], preferred_element_type=jnp.float32)
                    + b1_ref[...][None, :], 0.0)
    o = jnp.maximum(jnp.dot(o, w15_ref[...], preferred_element_type=jnp.float32)
                    + b15_ref[...][None, :], 0.0)
    out_ref[...] = (jnp.dot(o, w2_ref[...], preferred_element_type=jnp.float32)
                    + b2_ref[...][None, :])


def _mlp(sums, cnts, wm1, bm1, wm15, bm15, wm2, bm2):
    return pl.pallas_call(
        _mlp_body,
        out_shape=jax.ShapeDtypeStruct((_G, 256), jnp.float32),
    )(sums, cnts, wm1, bm1, wm15, bm15, wm2, bm2)


_SC_MESH = plsc.VectorSubcoreMesh(core_axis_name="c", subcore_axis_name="s")


@functools.partial(
    pl.kernel,
    mesh=_SC_MESH,
    out_type=jax.ShapeDtypeStruct((2 * _NP, _HALF), jnp.float32),
    scratch_types=[
        pltpu.VMEM((_NBUF, _CH), jnp.int32),
        pltpu.VMEM((_CPT, _CH), jnp.int32),
        pltpu.VMEM((_NBUF, _CH, _HALF), jnp.float32),
        pltpu.VMEM_SHARED((_NP, _HALF), jnp.float32),
        pltpu.SemaphoreType.DMA((_NBUF,)),
        pltpu.SemaphoreType.DMA((_NBUF,)),
        pltpu.SemaphoreType.DMA((_NBUF,)),
    ],
)
def _edge_scatter_sc(g_hbm, src3_hbm, dst3_hbm, z_hbm, out_hbm,
                     sidx, didx, rows, acc, isem, gsem, ssem):
    c = lax.axis_index("c")
    s = lax.axis_index("s")
    w = c * _TILES + s
    rbase = s * _RPT
    pltpu.sync_copy(z_hbm.at[pl.ds(rbase, _RPT)], acc.at[pl.ds(rbase, _RPT)])
    pltpu.sync_copy(dst3_hbm.at[s], didx)
    for b in range(_NBUF):
        pltpu.async_copy(src3_hbm.at[w].at[b], sidx.at[b], isem.at[b])
    plsc.subcore_barrier()

    def _wait_i(b):
        pltpu.make_async_copy(src3_hbm.at[w].at[0], sidx.at[b],
                              isem.at[b]).wait()

    def _wait_g(b):
        pltpu.make_async_copy(g_hbm.at[sidx.at[0]], rows.at[b],
                              gsem.at[b]).wait()

    def _wait_s(b):
        pltpu.make_async_copy(rows.at[b], acc.at[didx.at[0]],
                              ssem.at[b]).wait()

    for j in range(2):
        _wait_i(j)
        pltpu.async_copy(g_hbm.at[sidx.at[j]], rows.at[j], gsem.at[j])

    def rnd(t, carry):
        for br in range(_NBUF):
            j = _NBUF * t + 2 + br
            bj = (2 + br) % _NBUF
            _wait_i(bj)

            @pl.when(j >= _NBUF)
            def _w():
                _wait_s(bj)

            pltpu.async_copy(g_hbm.at[sidx.at[bj]], rows.at[bj], gsem.at[bj])
            _wait_g(br)
            pltpu.async_copy(rows.at[br], acc.at[didx.at[j - 2]],
                             ssem.at[br], add=True)

            @pl.when(j + 1 < _CPT)
            def _p():
                pltpu.async_copy(src3_hbm.at[w].at[j + 1], sidx.at[br],
                                 isem.at[br])
        return carry

    lax.fori_loop(0, _ROUNDS, rnd, 0)
    for j in range(_CPT - 2, _CPT):
        b = j % _NBUF
        _wait_g(b)
        pltpu.async_copy(rows.at[b], acc.at[didx.at[j]], ssem.at[b], add=True)
    for b in range(_NBUF):
        _wait_s(b)
    plsc.subcore_barrier()
    pltpu.sync_copy(acc.at[pl.ds(rbase, _RPT)],
                    out_hbm.at[pl.ds(c * _NP + rbase, _RPT)])


@functools.partial(
    pl.kernel,
    mesh=_SC_MESH,
    out_type=jax.ShapeDtypeStruct((2 * _NP,), jnp.float32),
    scratch_types=[
        pltpu.VMEM((_CPTD, _CHD), jnp.int32),
        pltpu.VMEM((_CHD,), jnp.float32),
        pltpu.VMEM_SHARED((_NP,), jnp.float32),
        pltpu.SemaphoreType.DMA,
    ],
)
def _deg_sc(dstd_hbm, zn_hbm, ones_hbm, out_hbm, didx, ones_v, acc, sem):
    c = lax.axis_index("c")
    s = lax.axis_index("s")
    rbase = s * _RPT
    pltpu.sync_copy(zn_hbm.at[pl.ds(rbase, _RPT)], acc.at[pl.ds(rbase, _RPT)])
    pltpu.sync_copy(ones_hbm, ones_v)
    pltpu.sync_copy(dstd_hbm.at[c * _TILES + s], didx)
    plsc.subcore_barrier()
    descs = [pltpu.async_copy(ones_v, acc.at[didx.at[j]], sem, add=True)
             for j in range(_CPTD)]
    for d in descs:
        d.wait()
    plsc.subcore_barrier()
    pltpu.sync_copy(acc.at[pl.ds(rbase, _RPT)],
                    out_hbm.at[pl.ds(c * _NP + rbase, _RPT)])


def _deg_partials(dstd, zn, ones):
    degflat = _deg_sc(dstd, zn, ones)
    return jnp.stack([degflat[:_N], degflat[_NP:_NP + _N]], axis=1)


def _edge_scatter(g, src3, dst3, z):
    sflat = _edge_scatter_sc(g.reshape(2 * _NP, _HALF), src3, dst3, z)
    return sflat.reshape(2, _NP, _HALF)


def kernel(x, edge_index, batch, W1, b1, W2, b2, W3, b3, W4, b4, W5, b5, W6, b6,
           Wm1, bm1, Wm15, bm15, Wm2, bm2):
    src, dst = lax.sort_key_val(edge_index[0], edge_index[1])
    batch2d = batch.reshape(_N, 1)

    epad = jnp.full((_TILES, _EPT - _E // _TILES), _NP - 1, jnp.int32)
    src_t = jnp.concatenate([src.reshape(_TILES, -1), epad], axis=1)
    src3 = jnp.concatenate([src_t, src_t + _NP], axis=0).reshape(
        2 * _TILES, _CPT, _CH)
    dst3 = jnp.concatenate([dst.reshape(_TILES, -1), epad], axis=1).reshape(
        _TILES, _CPT, _CH)
    dpad = jnp.full((2 * _TILES, _EPTD - _E // (2 * _TILES)), _NP - 1,
                    jnp.int32)
    dstd = jnp.concatenate([dst.reshape(2 * _TILES, -1), dpad],
                           axis=1).reshape(2 * _TILES, _CPTD, _CHD)
    z = jnp.zeros((_NP, _HALF), jnp.float32)
    zn = jnp.zeros((_NP,), jnp.float32)
    ones = jnp.ones((_CHD,), jnp.float32)

    degt = _deg_partials(dstd, zn, ones)
    g1, dinv = _first_layer(x, degt, W1)
    s1 = _edge_scatter(g1, src3, dst3, z)
    g2, h1 = _mid_layer(s1, g1, b1, dinv, W2, act_out=True)
    s2 = _edge_scatter(g2, src3, dst3, z)
    g3 = _mid_layer(s2, g2, b2, dinv, W3)
    s3 = _edge_scatter(g3, src3, dst3, z)
    g4, h = _mid_layer(s3, g3, b3, dinv, W4, res=h1, act_out=True)
    s4 = _edge_scatter(g4, src3, dst3, z)
    g5 = _mid_layer(s4, g4, b4, dinv, W5)
    s5 = _edge_scatter(g5, src3, dst3, z)
    g6 = _mid_layer(s5, g5, b5, dinv, W6, res=h)
    s6 = _edge_scatter(g6, src3, dst3, z)
    sums, cnts = _pool(s6, g6, b6, dinv, batch2d)
    return _mlp(sums, cnts, Wm1, bm1, Wm15, bm15, Wm2, bm2)


# speedup vs baseline: 1.3613x; 1.3613x over previous
import functools

import jax
import jax.numpy as jnp
from jax import lax
from jax.experimental import pallas as pl
from jax.experimental.pallas import tpu as pltpu
from jax.experimental.pallas import tpu_sc as plsc

_N = 10000
_E = 160000
_G = 64
_D = 256
_HALF = 128
_R = 1000
_GRID = _N // _R

_NP = 10240
_TILES = 16
_RPT = _NP // _TILES
_CH = 80
_EPT = _NP
_CPT = _EPT // _CH
_NBUF = 3
_ROUNDS = (_CPT - 2) // _NBUF
_CHD = 128
_EPTD = 5120
_CPTD = _EPTD // _CHD


def _first_body(x_ref, degt_ref, w_ref, g_ref, dinv_ref):
    d2 = degt_ref[...]
    deg = 1.0 + d2[:, 0:1] + d2[:, 1:2]
    dinv = lax.rsqrt(deg)
    gn = dinv * jnp.dot(x_ref[...], w_ref[...], preferred_element_type=jnp.float32)
    g_ref[0] = gn[:, :_HALF]
    g_ref[1] = gn[:, _HALF:]
    dinv_ref[...] = dinv


def _first_layer(x, degt, w1):
    return pl.pallas_call(
        _first_body,
        grid=(_GRID,),
        in_specs=[
            pl.BlockSpec((_R, _D), lambda i: (i, 0)),
            pl.BlockSpec((_R, 2), lambda i: (i, 0)),
            pl.BlockSpec((_D, _D), lambda i: (0, 0)),
        ],
        out_specs=[
            pl.BlockSpec((2, _R, _HALF), lambda i: (0, i, 0)),
            pl.BlockSpec((_R, 1), lambda i: (i, 0)),
        ],
        out_shape=[
            jax.ShapeDtypeStruct((2, _NP, _HALF), jnp.float32),
            jax.ShapeDtypeStruct((_N, 1), jnp.float32),
        ],
    )(x, degt, w1)


def _mid_body(s_ref, g_ref, b_ref, dinv_ref, w_ref, *rest, has_res, act_out):
    if has_res:
        res_ref = rest[0]
        rest = rest[1:]
    gn_ref = rest[0]
    dinv = dinv_ref[...]
    s_cat = jnp.concatenate([s_ref[0], s_ref[1]], axis=1)
    g_cat = jnp.concatenate([g_ref[0], g_ref[1]], axis=1)
    h = dinv * (s_cat + g_cat) + b_ref[...][None, :]
    act = jnp.maximum(h, 0.0)
    if has_res:
        act = act + res_ref[...]
    if act_out:
        rest[1][...] = act
    gn = dinv * jnp.dot(act, w_ref[...], preferred_element_type=jnp.float32)
    gn_ref[0] = gn[:, :_HALF]
    gn_ref[1] = gn[:, _HALF:]


def _mid_layer(s, g, b, dinv, w, res=None, act_out=False):
    has_res = res is not None
    in_specs = [
        pl.BlockSpec((2, _R, _HALF), lambda i: (0, i, 0)),
        pl.BlockSpec((2, _R, _HALF), lambda i: (0, i, 0)),
        pl.BlockSpec((_D,), lambda i: (0,)),
        pl.BlockSpec((_R, 1), lambda i: (i, 0)),
        pl.BlockSpec((_D, _D), lambda i: (0, 0)),
    ]
    args = [s, g, b, dinv, w]
    if has_res:
        in_specs.append(pl.BlockSpec((_R, _D), lambda i: (i, 0)))
        args.append(res)
    out_specs = [pl.BlockSpec((2, _R, _HALF), lambda i: (0, i, 0))]
    out_shape = [jax.ShapeDtypeStruct((2, _NP, _HALF), jnp.float32)]
    if act_out:
        out_specs.append(pl.BlockSpec((_R, _D), lambda i: (i, 0)))
        out_shape.append(jax.ShapeDtypeStruct((_N, _D), jnp.float32))
    out = pl.pallas_call(
        functools.partial(_mid_body, has_res=has_res, act_out=act_out),
        grid=(_GRID,),
        in_specs=in_specs,
        out_specs=out_specs,
        out_shape=out_shape,
    )(*args)
    return out if act_out else out[0]


def _pool_body(s_ref, g_ref, b_ref, dinv_ref, batch_ref, sums_ref, cnts_ref):
    i = pl.program_id(0)
    s_cat = jnp.concatenate([s_ref[0], s_ref[1]], axis=1)
    g_cat = jnp.concatenate([g_ref[0], g_ref[1]], axis=1)
    h = dinv_ref[...] * (s_cat + g_cat) + b_ref[...][None, :]
    gid = lax.broadcasted_iota(jnp.int32, (_R, _G), 1)
    mask = (batch_ref[...] == gid).astype(jnp.float32)
    part = lax.dot_general(mask, h, (((0,), (0,)), ((), ())),
                           preferred_element_type=jnp.float32)
    cpart = lax.dot_general(mask, jnp.ones((_R, 1), jnp.float32),
                            (((0,), (0,)), ((), ())),
                            preferred_element_type=jnp.float32)

    @pl.when(i == 0)
    def _init():
        sums_ref[...] = jnp.zeros_like(sums_ref)
        cnts_ref[...] = jnp.zeros_like(cnts_ref)

    sums_ref[...] += part
    cnts_ref[...] += cpart


def _pool(s, g, b, dinv, batch2d):
    return pl.pallas_call(
        _pool_body,
        grid=(_GRID,),
        in_specs=[
            pl.BlockSpec((2, _R, _HALF), lambda i: (0, i, 0)),
            pl.BlockSpec((2, _R, _HALF), lambda i: (0, i, 0)),
            pl.BlockSpec((_D,), lambda i: (0,)),
            pl.BlockSpec((_R, 1), lambda i: (i, 0)),
            pl.BlockSpec((_R, 1), lambda i: (i, 0)),
        ],
        out_specs=[
            pl.BlockSpec((_G, _D), lambda i: (0, 0)),
            pl.BlockSpec((_G, 1), lambda i: (0, 0)),
        ],
        out_shape=[
            jax.ShapeDtypeStruct((_G, _D), jnp.float32),
            jax.ShapeDtypeStruct((_G, 1), jnp.float32),
        ],
    )(s, g, b, dinv, batch2d)


def _mlp_body(sums_ref, cnts_ref, w1_ref, b1_ref, w15_ref, b15_ref, w2_ref,
              b2_ref, out_ref):
    pooled = sums_ref[...] / jnp.maximum(cnts_ref[...], 1.0)
    o = jnp.maximum(jnp.dot(pooled, w1_ref[...], preferred_element_type=jnp.float32)
                    + b1_ref[...][None, :], 0.0)
    o = jnp.maximum(jnp.dot(o, w15_ref[...], preferred_element_type=jnp.float32)
                    + b15_ref[...][None, :], 0.0)
    out_ref[...] = (jnp.dot(o, w2_ref[...], preferred_element_type=jnp.float32)
                    + b2_ref[...][None, :])


def _mlp(sums, cnts, wm1, bm1, wm15, bm15, wm2, bm2):
    return pl.pallas_call(
        _mlp_body,
        out_shape=jax.ShapeDtypeStruct((_G, 256), jnp.float32),
    )(sums, cnts, wm1, bm1, wm15, bm15, wm2, bm2)


_SC_MESH = plsc.VectorSubcoreMesh(core_axis_name="c", subcore_axis_name="s")


@functools.partial(
    pl.kernel,
    mesh=_SC_MESH,
    out_type=jax.ShapeDtypeStruct((2 * _NP, _HALF), jnp.float32),
    scratch_types=[
        pltpu.VMEM((_NBUF, _CH), jnp.int32),
        pltpu.VMEM((_CPT, _CH), jnp.int32),
        pltpu.VMEM((_NBUF, _CH, _HALF), jnp.float32),
        pltpu.VMEM_SHARED((_NP, _HALF), jnp.float32),
        pltpu.SemaphoreType.DMA((_NBUF,)),
        pltpu.SemaphoreType.DMA((_NBUF,)),
        pltpu.SemaphoreType.DMA((_NBUF,)),
    ],
)
def _edge_scatter_sc(g_hbm, src3_hbm, dst3_hbm, z_hbm, out_hbm,
                     sidx, didx, rows, acc, isem, gsem, ssem):
    c = lax.axis_index("c")
    s = lax.axis_index("s")
    w = c * _TILES + s
    rbase = s * _RPT
    pltpu.sync_copy(z_hbm.at[pl.ds(rbase, _RPT)], acc.at[pl.ds(rbase, _RPT)])
    pltpu.sync_copy(dst3_hbm.at[s], didx)
    for b in range(_NBUF):
        pltpu.async_copy(src3_hbm.at[w].at[b], sidx.at[b], isem.at[b])
    plsc.subcore_barrier()

    def _wait_i(b):
        pltpu.make_async_copy(src3_hbm.at[w].at[0], sidx.at[b],
                              isem.at[b]).wait()

    def _wait_g(b):
        pltpu.make_async_copy(g_hbm.at[sidx.at[0]], rows.at[b],
                              gsem.at[b]).wait()

    def _wait_s(b):
        pltpu.make_async_copy(rows.at[b], acc.at[didx.at[0]],
                              ssem.at[b]).wait()

    for j in range(2):
        _wait_i(j)
        pltpu.async_copy(g_hbm.at[sidx.at[j]], rows.at[j], gsem.at[j])

    def rnd(t, carry):
        for br in range(_NBUF):
            j = _NBUF * t + 2 + br
            bj = (2 + br) % _NBUF
            _wait_i(bj)

            @pl.when(j >= _NBUF)
            def _w():
                _wait_s(bj)

            pltpu.async_copy(g_hbm.at[sidx.at[bj]], rows.at[bj], gsem.at[bj])
            _wait_g(br)
            pltpu.async_copy(rows.at[br], acc.at[didx.at[j - 2]],
                             ssem.at[br], add=True)

            @pl.when(j + 1 < _CPT)
            def _p():
                pltpu.async_copy(src3_hbm.at[w].at[j + 1], sidx.at[br],
                                 isem.at[br])
        return carry

    lax.fori_loop(0, _ROUNDS, rnd, 0)
    for j in range(_CPT - 2, _CPT):
        b = j % _NBUF
        _wait_g(b)
        pltpu.async_copy(rows.at[b], acc.at[didx.at[j]], ssem.at[b], add=True)
    for b in range(_NBUF):
        _wait_s(b)
    plsc.subcore_barrier()
    pltpu.sync_copy(acc.at[pl.ds(rbase, _RPT)],
                    out_hbm.at[pl.ds(c * _NP + rbase, _RPT)])


@functools.partial(
    pl.kernel,
    mesh=_SC_MESH,
    out_type=jax.ShapeDtypeStruct((2 * _NP,), jnp.float32),
    scratch_types=[
        pltpu.VMEM((_CPTD, _CHD), jnp.int32),
        pltpu.VMEM((_CHD,), jnp.float32),
        pltpu.VMEM_SHARED((_NP,), jnp.float32),
        pltpu.SemaphoreType.DMA,
    ],
)
def _deg_sc(dstd_hbm, zn_hbm, ones_hbm, out_hbm, didx, ones_v, acc, sem):
    c = lax.axis_index("c")
    s = lax.axis_index("s")
    rbase = s * _RPT
    pltpu.sync_copy(zn_hbm.at[pl.ds(rbase, _RPT)], acc.at[pl.ds(rbase, _RPT)])
    pltpu.sync_copy(ones_hbm, ones_v)
    pltpu.sync_copy(dstd_hbm.at[c * _TILES + s], didx)
    plsc.subcore_barrier()
    descs = [pltpu.async_copy(ones_v, acc.at[didx.at[j]], sem, add=True)
             for j in range(_CPTD)]
    for d in descs:
        d.wait()
    plsc.subcore_barrier()
    pltpu.sync_copy(acc.at[pl.ds(rbase, _RPT)],
                    out_hbm.at[pl.ds(c * _NP + rbase, _RPT)])


def _deg_partials(dstd, zn, ones):
    degflat = _deg_sc(dstd, zn, ones)
    return jnp.stack([degflat[:_N], degflat[_NP:_NP + _N]], axis=1)


def _edge_scatter(g, src3, dst3, z):
    sflat = _edge_scatter_sc(g.reshape(2 * _NP, _HALF), src3, dst3, z)
    return sflat.reshape(2, _NP, _HALF)


def kernel(x, edge_index, batch, W1, b1, W2, b2, W3, b3, W4, b4, W5, b5, W6, b6,
           Wm1, bm1, Wm15, bm15, Wm2, bm2):
    src = edge_index[0]
    dst = edge_index[1]
    batch2d = batch.reshape(_N, 1)

    epad = jnp.full((_TILES, _EPT - _E // _TILES), _NP - 1, jnp.int32)
    src_t = jnp.concatenate([src.reshape(_TILES, -1), epad], axis=1)
    src3 = jnp.concatenate([src_t, src_t + _NP], axis=0).reshape(
        2 * _TILES, _CPT, _CH)
    dst3 = jnp.concatenate([dst.reshape(_TILES, -1), epad], axis=1).reshape(
        _TILES, _CPT, _CH)
    dpad = jnp.full((2 * _TILES, _EPTD - _E // (2 * _TILES)), _NP - 1,
                    jnp.int32)
    dstd = jnp.concatenate([dst.reshape(2 * _TILES, -1), dpad],
                           axis=1).reshape(2 * _TILES, _CPTD, _CHD)
    z = jnp.zeros((_NP, _HALF), jnp.float32)
    zn = jnp.zeros((_NP,), jnp.float32)
    ones = jnp.ones((_CHD,), jnp.float32)

    degt = _deg_partials(dstd, zn, ones)
    g1, dinv = _first_layer(x, degt, W1)
    s1 = _edge_scatter(g1, src3, dst3, z)
    g2, h1 = _mid_layer(s1, g1, b1, dinv, W2, act_out=True)
    s2 = _edge_scatter(g2, src3, dst3, z)
    g3 = _mid_layer(s2, g2, b2, dinv, W3)
    s3 = _edge_scatter(g3, src3, dst3, z)
    g4, h = _mid_layer(s3, g3, b3, dinv, W4, res=h1, act_out=True)
    s4 = _edge_scatter(g4, src3, dst3, z)
    g5 = _mid_layer(s4, g4, b4, dinv, W5)
    s5 = _edge_scatter(g5, src3, dst3, z)
    g6 = _mid_layer(s5, g5, b5, dinv, W6, res=h)
    s6 = _edge_scatter(g6, src3, dst3, z)
    sums, cnts = _pool(s6, g6, b6, dinv, batch2d)
    return _mlp(sums, cnts, Wm1, bm1, Wm15, bm15, Wm2, bm2)
